# baseline (speedup 1.0000x reference)
import functools

import jax
import jax.numpy as jnp
from jax import lax
from jax.experimental import pallas as pl
from jax.experimental.pallas import tpu as pltpu
from jax.experimental.pallas import tpu_sc as plsc

N_ENT = 1000000
IN_DIM = 128
HALF = IN_DIM // 2
BATCH = 16384
MARGIN = 6.0
EPSILON = 2.0
EMB_RANGE = (MARGIN + EPSILON) / IN_DIM
PI = 3.141592653589793
PHASE_SCALE = PI / EMB_RANGE

NC = 2
NS = 16
NW = NC * NS
B_PER_W = BATCH // NW
CHUNK = 128
N_CHUNKS = B_PER_W // CHUNK


def _sc_gather(h_idx, r_idx, t_idx, ent_embed, rel_embed):
    mesh = plsc.VectorSubcoreMesh(core_axis_name="c", subcore_axis_name="s")

    @functools.partial(
        pl.kernel,
        mesh=mesh,
        out_type=(
            jax.ShapeDtypeStruct((BATCH, IN_DIM), jnp.float32),
            jax.ShapeDtypeStruct((BATCH, IN_DIM), jnp.float32),
            jax.ShapeDtypeStruct((BATCH, HALF), jnp.float32),
        ),
        scratch_types=(
            pltpu.VMEM((CHUNK,), jnp.int32),
            pltpu.VMEM((CHUNK,), jnp.int32),
            pltpu.VMEM((CHUNK,), jnp.int32),
            pltpu.VMEM((CHUNK, IN_DIM), jnp.float32),
            pltpu.VMEM((CHUNK, IN_DIM), jnp.float32),
            pltpu.VMEM((CHUNK, HALF), jnp.float32),
            pltpu.SemaphoreType.DMA,
            pltpu.SemaphoreType.DMA,
        ),
    )
    def k(h_idx_hbm, r_idx_hbm, t_idx_hbm, ent_hbm, rel_hbm,
          head_out, tail_out, rel_out,
          hidx_v, ridx_v, tidx_v, head_v, tail_v, rel_v, sem, rsem):
        wid = lax.axis_index("s") * NC + lax.axis_index("c")
        base = wid * B_PER_W
        for c in range(N_CHUNKS):
            off = base + c * CHUNK
            pltpu.sync_copy(h_idx_hbm.at[pl.ds(off, CHUNK)], hidx_v)
            pltpu.sync_copy(t_idx_hbm.at[pl.ds(off, CHUNK)], tidx_v)
            pltpu.sync_copy(r_idx_hbm.at[pl.ds(off, CHUNK)], ridx_v)
            cp_h = pltpu.async_copy(ent_hbm.at[hidx_v], head_v, sem)
            cp_t = pltpu.async_copy(ent_hbm.at[tidx_v], tail_v, sem)

            def fire(g, _):
                idx16 = ridx_v[pl.ds(g * 16, 16)]
                for lane in range(16):
                    pltpu.async_copy(
                        rel_hbm.at[pl.ds(idx16[lane], 1)],
                        rel_v.at[pl.ds(g * 16 + lane, 1)], rsem)
                return 0
            lax.fori_loop(0, CHUNK // 16, fire, 0)
            pltpu.make_async_copy(
                rel_hbm.at[pl.ds(0, CHUNK)], rel_v, rsem).wait()
            cp_h.wait()
            cp_t.wait()
            pltpu.sync_copy(head_v, head_out.at[pl.ds(off, CHUNK)])
            pltpu.sync_copy(tail_v, tail_out.at[pl.ds(off, CHUNK)])
            pltpu.sync_copy(rel_v, rel_out.at[pl.ds(off, CHUNK)])

    return k(h_idx, r_idx, t_idx, ent_embed, rel_embed)


def _tc_math_body(head_ref, tail_ref, rel_ref, out_ref):
    head = head_ref[...]
    tail = tail_ref[...]
    phase = rel_ref[...] * PHASE_SCALE
    re_r = jnp.cos(phase)
    im_r = jnp.sin(phase)
    re_h = head[:, :HALF]
    im_h = head[:, HALF:]
    re_s = re_h * re_r - im_h * im_r - tail[:, :HALF]
    im_s = re_h * im_r + im_h * re_r - tail[:, HALF:]
    s = jnp.sqrt(re_s * re_s + im_s * im_s)
    out_ref[...] = MARGIN - jnp.sum(s, axis=1, keepdims=True)


def _tc_math(head, tail, rel):
    blk = 2048
    grid = (BATCH // blk,)
    return pl.pallas_call(
        _tc_math_body,
        grid=grid,
        in_specs=[
            pl.BlockSpec((blk, IN_DIM), lambda i: (i, 0)),
            pl.BlockSpec((blk, IN_DIM), lambda i: (i, 0)),
            pl.BlockSpec((blk, HALF), lambda i: (i, 0)),
        ],
        out_specs=pl.BlockSpec((blk, 1), lambda i: (i, 0)),
        out_shape=jax.ShapeDtypeStruct((BATCH, 1), jnp.float32),
    )(head, tail, rel)


def kernel(sample, ent_embed, rel_embed):
    h_idx = sample[:, 0]
    r_idx = sample[:, 1]
    t_idx = sample[:, 2]
    head, tail, rel = _sc_gather(h_idx, r_idx, t_idx, ent_embed, rel_embed)
    return _tc_math(head, tail, rel)
